# baseline (speedup 1.0000x reference)
import functools

import jax
import jax.numpy as jnp
from jax import lax
from jax.experimental import pallas as pl
from jax.experimental.pallas import tpu as pltpu
from jax.experimental.pallas import tpu_sc as plsc

_NC, _NS, _L = 2, 16, 16
_NW = _NC * _NS
_R_SC = 2048
_ROWS_W = _R_SC // _NW
_GROUPS = _ROWS_W // _L
_BB_TC = 1024


def _sc_body(state_hbm, lid_hbm, latents_hbm, wt_hbm, b_hbm, out_hbm,
             x_sh, x_v, wt_v, b_v, lat_v, out_v, idx_v, sem):
    cid = lax.axis_index("c")
    sid = lax.axis_index("s")
    wid = sid * _NC + cid
    base = wid * _ROWS_W
    dim_state = wt_hbm.shape[1]
    rows_sc = _R_SC // _NC
    rows_rnd = x_sh.shape[0]
    rows_tec_rnd = rows_rnd // _NS

    pltpu.sync_copy(wt_hbm, wt_v)
    pltpu.sync_copy(b_hbm, b_v.at[pl.ds(0, 4)])
    pltpu.sync_copy(lid_hbm, idx_v)
    pltpu.async_copy(latents_hbm.at[idx_v], lat_v, sem).wait()

    for k in range(rows_sc // rows_rnd):
        @pl.when(sid == 0)
        def _stage(k=k):
            pltpu.sync_copy(
                state_hbm.at[pl.ds(cid * rows_sc + k * rows_rnd, rows_rnd)],
                x_sh)
        plsc.subcore_barrier()
        pltpu.sync_copy(
            x_sh.at[pl.ds(sid * rows_tec_rnd, rows_tec_rnd)],
            x_v.at[pl.ds(k * rows_tec_rnd, rows_tec_rnd), pl.ds(0, dim_state)])
        plsc.subcore_barrier()

    _BISECT_DMA_ONLY = True
    if _BISECT_DMA_ONLY:
        for g in range(_GROUPS):
            for r in range(_L):
                row = g * _L + r
                for c in range(8):
                    out_v[row, pl.ds(c * _L, _L)] = b_v[...]
        pltpu.sync_copy(out_v, out_hbm.at[pl.ds(base, _ROWS_W)])
        return

    iota = lax.iota(jnp.int32, _L)
    row_ids = [iota + g * _L for g in range(_GROUPS)]
    bvec = b_v[...]
    bs = [bvec[n] for n in range(4)]

    def k_chunk(jc, accs):
        j0 = jc * _L
        wchunks = [wt_v[n, pl.ds(j0, _L)] for n in range(4)]
        accs = list(accs)
        for jj in range(_L):
            col = jnp.full((_L,), j0 + jj, jnp.int32)
            xcols = [plsc.load_gather(x_v, [row_ids[g], col])
                     for g in range(_GROUPS)]
            for g in range(_GROUPS):
                for n in range(4):
                    accs[g * 4 + n] = accs[g * 4 + n] + xcols[g] * wchunks[n][jj]
        return tuple(accs)

    accs = lax.fori_loop(
        0, dim_state // _L, k_chunk,
        tuple(jnp.zeros((_L,), jnp.float32) for _ in range(_GROUPS * 4)))

    lats = [[lat_v[0, n, pl.ds(c * _L, _L)] for c in range(8)]
            for n in range(4)]
    for g in range(_GROUPS):
        es = [jnp.exp(accs[g * 4 + n] + bs[n]) for n in range(4)]
        inv = 1.0 / (es[0] + es[1] + es[2] + es[3])
        gates = [es[n] * inv for n in range(4)]
        for r in range(_L):
            gs = [gates[n][r] for n in range(4)]
            row = g * _L + r
            for c in range(8):
                acc = gs[0] * lats[0][c]
                for n in range(1, 4):
                    acc = acc + gs[n] * lats[n][c]
                out_v[row, pl.ds(c * _L, _L)] = acc

    pltpu.sync_copy(out_v, out_hbm.at[pl.ds(base, _ROWS_W)])


def _sc_half(state, lid, latents, Wt, b):
    num_latents, num_sets, dim_latent = latents.shape
    dim_state = state.shape[1]
    return pl.kernel(
        _sc_body,
        out_type=jax.ShapeDtypeStruct((_R_SC, dim_latent), jnp.float32),
        mesh=plsc.VectorSubcoreMesh(core_axis_name="c", subcore_axis_name="s"),
        scratch_types=[
            pltpu.VMEM_SHARED((_R_SC // _NC // 2, dim_state),
                              jnp.float32),
            pltpu.VMEM((_ROWS_W, dim_state + 1), jnp.float32),
            pltpu.VMEM((num_sets, dim_state), jnp.float32),
            pltpu.VMEM((_L,), jnp.float32),
            pltpu.VMEM((1, num_sets, dim_latent), jnp.float32),
            pltpu.VMEM((_ROWS_W, dim_latent), jnp.float32),
            pltpu.VMEM((1,), jnp.int32),
            pltpu.SemaphoreType.DMA,
        ],
        compiler_params=pltpu.CompilerParams(
            needs_layout_passes=False, use_tc_tiling_on_sc=False),
    )(state, lid, latents, Wt, b)


def _tc_kernel(lid_ref, state_ref, latents_ref, w_ref, b_ref, out_ref):
    x = state_ref[...]
    w = w_ref[...]
    logits = jnp.dot(x, w, preferred_element_type=jnp.float32) + b_ref[...]
    e = jnp.exp(logits)
    gates = e / jnp.sum(e, axis=-1, keepdims=True)
    lat = latents_ref[0]
    out_ref[...] = jnp.dot(gates, lat, preferred_element_type=jnp.float32)


def _tc_half(state, lid, latents, W, b):
    batch, dim_state = state.shape
    _, num_sets, dim_latent = latents.shape
    rows = batch - _R_SC
    base_blk = _R_SC // _BB_TC

    return pl.pallas_call(
        _tc_kernel,
        grid_spec=pltpu.PrefetchScalarGridSpec(
            num_scalar_prefetch=1,
            grid=(rows // _BB_TC,),
            in_specs=[
                pl.BlockSpec((_BB_TC, dim_state),
                             lambda i, lid_ref: (base_blk + i, 0)),
                pl.BlockSpec((1, num_sets, dim_latent),
                             lambda i, lid_ref: (lid_ref[0], 0, 0)),
                pl.BlockSpec((dim_state, num_sets), lambda i, lid_ref: (0, 0)),
                pl.BlockSpec((num_sets,), lambda i, lid_ref: (0,)),
            ],
            out_specs=pl.BlockSpec((_BB_TC, dim_latent),
                                   lambda i, lid_ref: (i, 0)),
        ),
        out_shape=jax.ShapeDtypeStruct((rows, dim_latent), jnp.float32),
        compiler_params=pltpu.CompilerParams(
            dimension_semantics=("parallel",),
        ),
    )(lid, state, latents, W, b)


def kernel(state, latent_id, latents, W, b):
    lid = jnp.asarray(latent_id, jnp.int32).reshape(1)
    out_sc = _sc_half(state, lid, latents, W.T, b)
    out_tc = _tc_half(state, lid, latents, W, b)
    return jnp.concatenate([out_sc, out_tc], axis=0)


# speedup vs baseline: 1.2292x; 1.2292x over previous
import functools

import jax
import jax.numpy as jnp
from jax import lax
from jax.experimental import pallas as pl
from jax.experimental.pallas import tpu as pltpu
from jax.experimental.pallas import tpu_sc as plsc

_NC, _NS, _L = 2, 16, 16
_NW = _NC * _NS
_R_SC = 2048
_ROWS_W = _R_SC // _NW
_GROUPS = _ROWS_W // _L
_BB_TC = 1024


def _sc_body(state_hbm, lid_hbm, latents_hbm, wt_hbm, b_hbm, out_hbm,
             x_sh, x_v, wt_v, b_v, lat_v, out_v, idx_v, sem):
    cid = lax.axis_index("c")
    sid = lax.axis_index("s")
    wid = sid * _NC + cid
    base = wid * _ROWS_W
    dim_state = wt_hbm.shape[1]
    rows_sc = _R_SC // _NC
    rows_rnd = x_sh.shape[0]
    rows_tec_rnd = rows_rnd // _NS

    pltpu.sync_copy(state_hbm.at[pl.ds(base, _ROWS_W)],
                    x_v.at[:, pl.ds(0, dim_state)])

    _BISECT_DMA_ONLY = True
    if _BISECT_DMA_ONLY:
        pltpu.sync_copy(out_v, out_hbm.at[pl.ds(base, _ROWS_W)])
        return

    iota = lax.iota(jnp.int32, _L)
    row_ids = [iota + g * _L for g in range(_GROUPS)]
    bvec = b_v[...]
    bs = [bvec[n] for n in range(4)]

    def k_chunk(jc, accs):
        j0 = jc * _L
        wchunks = [wt_v[n, pl.ds(j0, _L)] for n in range(4)]
        accs = list(accs)
        for jj in range(_L):
            col = jnp.full((_L,), j0 + jj, jnp.int32)
            xcols = [plsc.load_gather(x_v, [row_ids[g], col])
                     for g in range(_GROUPS)]
            for g in range(_GROUPS):
                for n in range(4):
                    accs[g * 4 + n] = accs[g * 4 + n] + xcols[g] * wchunks[n][jj]
        return tuple(accs)

    accs = lax.fori_loop(
        0, dim_state // _L, k_chunk,
        tuple(jnp.zeros((_L,), jnp.float32) for _ in range(_GROUPS * 4)))

    lats = [[lat_v[0, n, pl.ds(c * _L, _L)] for c in range(8)]
            for n in range(4)]
    for g in range(_GROUPS):
        es = [jnp.exp(accs[g * 4 + n] + bs[n]) for n in range(4)]
        inv = 1.0 / (es[0] + es[1] + es[2] + es[3])
        gates = [es[n] * inv for n in range(4)]
        for r in range(_L):
            gs = [gates[n][r] for n in range(4)]
            row = g * _L + r
            for c in range(8):
                acc = gs[0] * lats[0][c]
                for n in range(1, 4):
                    acc = acc + gs[n] * lats[n][c]
                out_v[row, pl.ds(c * _L, _L)] = acc

    pltpu.sync_copy(out_v, out_hbm.at[pl.ds(base, _ROWS_W)])


def _sc_half(state, lid, latents, Wt, b):
    num_latents, num_sets, dim_latent = latents.shape
    dim_state = state.shape[1]
    return pl.kernel(
        _sc_body,
        out_type=jax.ShapeDtypeStruct((_R_SC, dim_latent), jnp.float32),
        mesh=plsc.VectorSubcoreMesh(core_axis_name="c", subcore_axis_name="s"),
        scratch_types=[
            pltpu.VMEM_SHARED((_R_SC // _NC // 2, dim_state),
                              jnp.float32),
            pltpu.VMEM((_ROWS_W, dim_state + 1), jnp.float32),
            pltpu.VMEM((num_sets, dim_state), jnp.float32),
            pltpu.VMEM((_L,), jnp.float32),
            pltpu.VMEM((1, num_sets, dim_latent), jnp.float32),
            pltpu.VMEM((_ROWS_W, dim_latent), jnp.float32),
            pltpu.VMEM((1,), jnp.int32),
            pltpu.SemaphoreType.DMA,
        ],
        compiler_params=pltpu.CompilerParams(
            needs_layout_passes=False, use_tc_tiling_on_sc=False),
    )(state, lid, latents, Wt, b)


def _tc_kernel(lid_ref, state_ref, latents_ref, w_ref, b_ref, out_ref):
    x = state_ref[...]
    w = w_ref[...]
    logits = jnp.dot(x, w, preferred_element_type=jnp.float32) + b_ref[...]
    e = jnp.exp(logits)
    gates = e / jnp.sum(e, axis=-1, keepdims=True)
    lat = latents_ref[0]
    out_ref[...] = jnp.dot(gates, lat, preferred_element_type=jnp.float32)


def _tc_half(state, lid, latents, W, b):
    batch, dim_state = state.shape
    _, num_sets, dim_latent = latents.shape
    rows = batch - _R_SC
    base_blk = _R_SC // _BB_TC

    return pl.pallas_call(
        _tc_kernel,
        grid_spec=pltpu.PrefetchScalarGridSpec(
            num_scalar_prefetch=1,
            grid=(rows // _BB_TC,),
            in_specs=[
                pl.BlockSpec((_BB_TC, dim_state),
                             lambda i, lid_ref: (base_blk + i, 0)),
                pl.BlockSpec((1, num_sets, dim_latent),
                             lambda i, lid_ref: (lid_ref[0], 0, 0)),
                pl.BlockSpec((dim_state, num_sets), lambda i, lid_ref: (0, 0)),
                pl.BlockSpec((num_sets,), lambda i, lid_ref: (0,)),
            ],
            out_specs=pl.BlockSpec((_BB_TC, dim_latent),
                                   lambda i, lid_ref: (i, 0)),
        ),
        out_shape=jax.ShapeDtypeStruct((rows, dim_latent), jnp.float32),
        compiler_params=pltpu.CompilerParams(
            dimension_semantics=("parallel",),
        ),
    )(lid, state, latents, W, b)


def kernel(state, latent_id, latents, W, b):
    lid = jnp.asarray(latent_id, jnp.int32).reshape(1)
    out_sc = _sc_half(state, lid, latents, W.T, b)
    out_tc = _tc_half(state, lid, latents, W, b)
    return jnp.concatenate([out_sc, out_tc], axis=0)


# speedup vs baseline: 5.0518x; 4.1099x over previous
import jax
import jax.numpy as jnp
from jax.experimental import pallas as pl
from jax.experimental.pallas import tpu as pltpu

_BB = 2048


def _fused_kernel(lid_ref, state_ref, latents_ref, w_ref, b_ref, out_ref):
    x = state_ref[...]
    w = w_ref[...]
    logits = jnp.dot(x, w, preferred_element_type=jnp.float32) + b_ref[...]
    e = jnp.exp(logits)
    gates = e / jnp.sum(e, axis=-1, keepdims=True)
    lat = latents_ref[0]
    out_ref[...] = jnp.dot(gates, lat, preferred_element_type=jnp.float32)


def kernel(state, latent_id, latents, W, b):
    batch, dim_state = state.shape
    _, num_sets, dim_latent = latents.shape

    grid = (batch // _BB,)
    out = pl.pallas_call(
        _fused_kernel,
        grid_spec=pltpu.PrefetchScalarGridSpec(
            num_scalar_prefetch=1,
            grid=grid,
            in_specs=[
                pl.BlockSpec((_BB, dim_state), lambda i, lid_ref: (i, 0)),
                pl.BlockSpec((1, num_sets, dim_latent),
                             lambda i, lid_ref: (lid_ref[0], 0, 0)),
                pl.BlockSpec((dim_state, num_sets), lambda i, lid_ref: (0, 0)),
                pl.BlockSpec((num_sets,), lambda i, lid_ref: (0,)),
            ],
            out_specs=pl.BlockSpec((_BB, dim_latent), lambda i, lid_ref: (i, 0)),
        ),
        out_shape=jax.ShapeDtypeStruct((batch, dim_latent), jnp.float32),
        compiler_params=pltpu.CompilerParams(
            dimension_semantics=("parallel",),
        ),
    )(jnp.asarray(latent_id, jnp.int32).reshape(1), state, latents, W, b)
    return out
